# Initial kernel scaffold; baseline (speedup 1.0000x reference)
import jax
import jax.numpy as jnp
from jax.experimental import pallas as pl


def kernel(x, sample_map):
    raise NotImplementedError("write your pallas kernel here")


# speedup vs baseline: 1.7912x; 1.7912x over previous
import functools

import jax
import jax.numpy as jnp
from jax import lax
from jax.experimental import pallas as pl
from jax.experimental.pallas import tpu as pltpu
from jax.experimental.pallas import tpu_sc as plsc

B, C, H, W = 2, 384, 224, 224
OH, OW, K = 112, 112, 4
BC = B * C
HW = H * W
OP = OH * OW

NC, NS = 2, 16
NW = NC * NS
PLANES_PER_W = BC // NW
P = 2
NBATCH = PLANES_PER_W // P
NCHUNK = 16
CHUNK = OP // NCHUNK
NVREG = CHUNK // 16


def _body(x_hbm, sm_hbm, out_hbm, pbuf0, pbuf1, smbuf, obuf0, obuf1):
    wid = lax.axis_index("s") * NC + lax.axis_index("c")
    plane_base = wid * PLANES_PER_W

    def batch_body(b, _):
        p0 = plane_base + b * P
        pltpu.sync_copy(x_hbm.at[pl.ds(p0 * HW, HW)], pbuf0)
        pltpu.sync_copy(x_hbm.at[pl.ds((p0 + 1) * HW, HW)], pbuf1)

        def chunk_body(c, _):
            pltpu.sync_copy(sm_hbm.at[pl.ds(c * (2 * K * CHUNK), 2 * K * CHUNK)], smbuf)

            def vreg_body(q, _):
                base = q * 16
                acc0 = jnp.zeros((16,), jnp.float32)
                acc1 = jnp.zeros((16,), jnp.float32)
                for k in range(K):
                    xs = smbuf[pl.ds((2 * k) * CHUNK + base, 16)]
                    ys = smbuf[pl.ds((2 * k + 1) * CHUNK + base, 16)]
                    x0 = jnp.clip(xs.astype(jnp.int32), 0, W - 2)
                    y0 = jnp.clip(ys.astype(jnp.int32), 0, H - 2)
                    wx = xs - x0.astype(jnp.float32)
                    wy = ys - y0.astype(jnp.float32)
                    i00 = y0 * W + x0
                    for pbuf, which in ((pbuf0, 0), (pbuf1, 1)):
                        v00 = plsc.load_gather(pbuf, [i00])
                        v01 = plsc.load_gather(pbuf, [i00 + 1])
                        v10 = plsc.load_gather(pbuf, [i00 + W])
                        v11 = plsc.load_gather(pbuf, [i00 + (W + 1)])
                        t0 = v00 + wx * (v01 - v00)
                        t1 = v10 + wx * (v11 - v10)
                        val = t0 + wy * (t1 - t0)
                        if which == 0:
                            acc0 = acc0 + val
                        else:
                            acc1 = acc1 + val
                obuf0[pl.ds(base, 16)] = acc0 * 0.25
                obuf1[pl.ds(base, 16)] = acc1 * 0.25
                return 0

            lax.fori_loop(0, NVREG, vreg_body, 0)
            pltpu.sync_copy(obuf0, out_hbm.at[pl.ds(p0 * OP + c * CHUNK, CHUNK)])
            pltpu.sync_copy(obuf1, out_hbm.at[pl.ds((p0 + 1) * OP + c * CHUNK, CHUNK)])
            return 0

        lax.fori_loop(0, NCHUNK, chunk_body, 0)
        return 0

    lax.fori_loop(0, NBATCH, batch_body, 0)


@jax.jit
def _mapped_avg_pool_sc(x2d, sm_t):
    k = pl.kernel(
        _body,
        out_type=jax.ShapeDtypeStruct((BC * OP,), jnp.float32),
        mesh=plsc.VectorSubcoreMesh(core_axis_name="c", subcore_axis_name="s"),
        scratch_types=[
            pltpu.VMEM((HW,), jnp.float32),
            pltpu.VMEM((HW,), jnp.float32),
            pltpu.VMEM((8 * CHUNK,), jnp.float32),
            pltpu.VMEM((CHUNK,), jnp.float32),
            pltpu.VMEM((CHUNK,), jnp.float32),
        ],
        compiler_params=pltpu.CompilerParams(needs_layout_passes=False),
    )
    return k(x2d, sm_t)


def kernel(x, sample_map):
    x1d = x.reshape(BC * HW)
    smf = sample_map.reshape(NCHUNK, CHUNK, 2 * K)
    sm_t = smf.transpose(0, 2, 1).reshape(NCHUNK * 2 * K * CHUNK)
    out1d = _mapped_avg_pool_sc(x1d, sm_t)
    return out1d.reshape(B, C, OH, OW)


# speedup vs baseline: 2.1454x; 1.1977x over previous
import jax
import jax.numpy as jnp
from jax import lax
from jax.experimental import pallas as pl
from jax.experimental.pallas import tpu as pltpu
from jax.experimental.pallas import tpu_sc as plsc

B, C, H, W = 2, 384, 224, 224
OH, OW, K = 112, 112, 4
BC = B * C
HW = H * W
OP = OH * OW

NC, NS = 2, 16
NW = NC * NS
PLANES_PER_W = BC // NW
P = 2
NBATCH = PLANES_PER_W // P
NCHUNK = 16
CHUNK = OP // NCHUNK
SMROW = 2 * K * CHUNK
NPAIR = NCHUNK // 2


def _body(x_hbm, sm_hbm, out_hbm,
          pbuf0, pbuf1, sma, smb, oa0, oa1, ob0, ob1,
          sm_sem_a, sm_sem_b, out_sem_a, out_sem_b):
    wid = lax.axis_index("s") * NC + lax.axis_index("c")
    plane_base = wid * PLANES_PER_W

    def compute_chunk(smbuf, obuf0, obuf1):
        @plsc.parallel_loop(0, CHUNK, step=16, unroll=2)
        def _(base):
            acc0 = jnp.zeros((16,), jnp.float32)
            acc1 = jnp.zeros((16,), jnp.float32)
            for k in range(K):
                xs = smbuf[pl.ds((2 * k) * CHUNK + base, 16)]
                ys = smbuf[pl.ds((2 * k + 1) * CHUNK + base, 16)]
                x0 = jnp.clip(xs.astype(jnp.int32), 0, W - 2)
                y0 = jnp.clip(ys.astype(jnp.int32), 0, H - 2)
                wx = xs - x0.astype(jnp.float32)
                wy = ys - y0.astype(jnp.float32)
                i00 = y0 * W + x0
                for pbuf, which in ((pbuf0, 0), (pbuf1, 1)):
                    v00 = plsc.load_gather(pbuf, [i00])
                    v01 = plsc.load_gather(pbuf, [i00 + 1])
                    v10 = plsc.load_gather(pbuf, [i00 + W])
                    v11 = plsc.load_gather(pbuf, [i00 + (W + 1)])
                    t0 = v00 + wx * (v01 - v00)
                    t1 = v10 + wx * (v11 - v10)
                    val = t0 + wy * (t1 - t0)
                    if which == 0:
                        acc0 = acc0 + val
                    else:
                        acc1 = acc1 + val
            obuf0[pl.ds(base, 16)] = acc0 * 0.25
            obuf1[pl.ds(base, 16)] = acc1 * 0.25

    def sm_start(c, buf, sem):
        pltpu.async_copy(sm_hbm.at[pl.ds(c * SMROW, SMROW)], buf, sem)

    def sm_wait(buf, sem):
        pltpu.make_async_copy(sm_hbm.at[pl.ds(0, SMROW)], buf, sem).wait()

    def out_start(p0, c, obuf0, obuf1, sem):
        pltpu.async_copy(obuf0, out_hbm.at[pl.ds(p0 * OP + c * CHUNK, CHUNK)], sem)
        pltpu.async_copy(obuf1, out_hbm.at[pl.ds((p0 + 1) * OP + c * CHUNK, CHUNK)], sem)

    def out_drain(obuf0, obuf1, sem):
        pltpu.make_async_copy(out_hbm.at[pl.ds(0, CHUNK)], obuf0, sem).wait()
        pltpu.make_async_copy(out_hbm.at[pl.ds(0, CHUNK)], obuf1, sem).wait()

    def batch_body(b, _):
        p0 = plane_base + b * P
        pltpu.sync_copy(x_hbm.at[pl.ds(p0 * HW, HW)], pbuf0)
        pltpu.sync_copy(x_hbm.at[pl.ds((p0 + 1) * HW, HW)], pbuf1)
        sm_start(0, sma, sm_sem_a)
        sm_start(1, smb, sm_sem_b)

        def pair_body(ci, _):
            c0 = ci * 2
            sm_wait(sma, sm_sem_a)

            @pl.when(ci > 0)
            def _():
                out_drain(oa0, oa1, out_sem_a)

            compute_chunk(sma, oa0, oa1)
            out_start(p0, c0, oa0, oa1, out_sem_a)

            @pl.when(ci < NPAIR - 1)
            def _():
                sm_start(c0 + 2, sma, sm_sem_a)

            sm_wait(smb, sm_sem_b)

            @pl.when(ci > 0)
            def _():
                out_drain(ob0, ob1, out_sem_b)

            compute_chunk(smb, ob0, ob1)
            out_start(p0, c0 + 1, ob0, ob1, out_sem_b)

            @pl.when(ci < NPAIR - 1)
            def _():
                sm_start(c0 + 3, smb, sm_sem_b)

            return 0

        lax.fori_loop(0, NPAIR, pair_body, 0)
        out_drain(oa0, oa1, out_sem_a)
        out_drain(ob0, ob1, out_sem_b)
        return 0

    lax.fori_loop(0, NBATCH, batch_body, 0)


@jax.jit
def _mapped_avg_pool_sc(x1d, sm_t):
    k = pl.kernel(
        _body,
        out_type=jax.ShapeDtypeStruct((BC * OP,), jnp.float32),
        mesh=plsc.VectorSubcoreMesh(core_axis_name="c", subcore_axis_name="s"),
        scratch_types=[
            pltpu.VMEM((HW,), jnp.float32),
            pltpu.VMEM((HW,), jnp.float32),
            pltpu.VMEM((SMROW,), jnp.float32),
            pltpu.VMEM((SMROW,), jnp.float32),
            pltpu.VMEM((CHUNK,), jnp.float32),
            pltpu.VMEM((CHUNK,), jnp.float32),
            pltpu.VMEM((CHUNK,), jnp.float32),
            pltpu.VMEM((CHUNK,), jnp.float32),
            pltpu.SemaphoreType.DMA,
            pltpu.SemaphoreType.DMA,
            pltpu.SemaphoreType.DMA,
            pltpu.SemaphoreType.DMA,
        ],
        compiler_params=pltpu.CompilerParams(needs_layout_passes=False),
    )
    return k(x1d, sm_t)


def kernel(x, sample_map):
    x1d = x.reshape(BC * HW)
    smf = sample_map.reshape(NCHUNK, CHUNK, 2 * K)
    sm_t = smf.transpose(0, 2, 1).reshape(NCHUNK * SMROW)
    out1d = _mapped_avg_pool_sc(x1d, sm_t)
    return out1d.reshape(B, C, OH, OW)
